# baseline (speedup 1.0000x reference)
import jax
import jax.numpy as jnp
from jax.experimental import pallas as pl
from jax.experimental.pallas import tpu as pltpu

SBLK = 8192
CH = 256
NCH = SBLK // CH
QH = 8
P_DIM = 64
N_DIM = 16


def _hi_lo(x):
    hi = x.astype(jnp.bfloat16)
    lo = (x - hi.astype(jnp.float32)).astype(jnp.bfloat16)
    return hi, lo


_MM = (((1,), (0,)), ((), ()))
_KK = (((1,), (1,)), ((), ()))


def _mask_dot(val_f32, mask_bf16):
    hi, lo = _hi_lo(val_f32)
    return (
        jax.lax.dot_general(hi, mask_bf16, _MM, preferred_element_type=jnp.float32)
        + jax.lax.dot_general(lo, mask_bf16, _MM, preferred_element_type=jnp.float32)
    )


def _fused_kernel(a_ref, x_ref, b_ref, su_ref, e_ref, o_ref, carry_ref, acc_ref):
    s = pl.program_id(1)
    ns = pl.num_programs(1)

    @pl.when(s == 0)
    def _():
        carry_ref[...] = jnp.zeros_like(carry_ref)

    a = a_ref[0]
    su = su_ref[...]

    stk = jnp.concatenate(
        [a[:, c * CH:(c + 1) * CH] for c in range(NCH)], axis=0)
    rest_stk = _mask_dot(stk, su)
    tot_stk = jnp.sum(stk, axis=1, keepdims=True)

    suffix = carry_ref[...]
    rests = [None] * NCH
    for c in reversed(range(NCH)):
        rests[c] = rest_stk[c * QH:(c + 1) * QH] + suffix
        suffix = suffix + tot_stk[c * QH:(c + 1) * QH]
    carry_ref[...] = suffix

    w = jnp.exp(jnp.concatenate(rests, axis=1))
    whi, wlo = _hi_lo(w)
    e = e_ref[...]
    wg = (jax.lax.dot_general(e, whi, _MM, preferred_element_type=jnp.float32)
          + jax.lax.dot_general(e, wlo, _MM, preferred_element_type=jnp.float32))

    xb = x_ref[0].astype(jnp.bfloat16)
    bwb = (b_ref[0] * wg).astype(jnp.bfloat16)
    z = jax.lax.dot_general(
        xb, bwb, _KK, preferred_element_type=jnp.float32)

    @pl.when(s == 0)
    def _():
        acc_ref[...] = z

    @pl.when(s != 0)
    def _():
        acc_ref[...] = acc_ref[...] + z

    @pl.when(s == ns - 1)
    def _():
        acc = acc_ref[...]
        for hl in range(QH):
            o_ref[0, hl] = acc[hl * P_DIM:(hl + 1) * P_DIM,
                               hl * N_DIM:(hl + 1) * N_DIM]


def kernel(X, A, B_mat):
    B, S, H, P = X.shape
    N = B_mat.shape[-1]
    nq = H // QH
    ns = S // SBLK
    xw = QH * P
    bw = QH * N

    kk = jax.lax.broadcasted_iota(jnp.int32, (CH, CH), 0)
    ll = jax.lax.broadcasted_iota(jnp.int32, (CH, CH), 1)
    su = (kk > ll).astype(jnp.bfloat16)
    rr = jax.lax.broadcasted_iota(jnp.int32, (bw, QH), 0)
    hh = jax.lax.broadcasted_iota(jnp.int32, (bw, QH), 1)
    e = (rr // N == hh).astype(jnp.bfloat16)

    Xt = X.transpose(0, 2, 3, 1).reshape(B, H * P, S)
    Bt = B_mat.transpose(0, 2, 3, 1).reshape(B, H * N, S)
    At = A.transpose(0, 2, 1)

    out_full = pl.pallas_call(
        _fused_kernel,
        grid=(B * nq, ns),
        in_specs=[
            pl.BlockSpec((1, QH, SBLK), lambda i, s: (i // nq, i % nq, ns - 1 - s)),
            pl.BlockSpec((1, xw, SBLK), lambda i, s: (i // nq, i % nq, ns - 1 - s)),
            pl.BlockSpec((1, bw, SBLK), lambda i, s: (i // nq, i % nq, ns - 1 - s)),
            pl.BlockSpec((CH, CH), lambda i, s: (0, 0)),
            pl.BlockSpec((bw, QH), lambda i, s: (0, 0)),
        ],
        out_specs=pl.BlockSpec((1, QH, P, N), lambda i, s: (i, 0, 0, 0)),
        out_shape=jax.ShapeDtypeStruct((B * nq, QH, P, N), jnp.float32),
        scratch_shapes=[
            pltpu.VMEM((QH, 1), jnp.float32),
            pltpu.VMEM((xw, bw), jnp.float32),
        ],
        compiler_params=pltpu.CompilerParams(
            dimension_semantics=("parallel", "arbitrary")),
    )(At, Xt, Bt, su, e)

    return out_full.reshape(B, H, P, N)


# speedup vs baseline: 1.0364x; 1.0364x over previous
import jax
import jax.numpy as jnp
from jax.experimental import pallas as pl
from jax.experimental.pallas import tpu as pltpu

SBLK = 4096
CH = 256
NCH = SBLK // CH
QH = 8
P_DIM = 64
N_DIM = 16


def _hi_lo(x):
    hi = x.astype(jnp.bfloat16)
    lo = (x - hi.astype(jnp.float32)).astype(jnp.bfloat16)
    return hi, lo


_MM = (((1,), (0,)), ((), ()))
_KK = (((1,), (1,)), ((), ()))


def _mask_dot(val_f32, mask_bf16):
    hi, lo = _hi_lo(val_f32)
    return (
        jax.lax.dot_general(hi, mask_bf16, _MM, preferred_element_type=jnp.float32)
        + jax.lax.dot_general(lo, mask_bf16, _MM, preferred_element_type=jnp.float32)
    )


def _fused_kernel(a_ref, x_ref, b_ref, su_ref, e_ref, o_ref, carry_ref, acc_ref):
    s = pl.program_id(1)
    ns = pl.num_programs(1)

    @pl.when(s == 0)
    def _():
        carry_ref[...] = jnp.zeros_like(carry_ref)

    a = a_ref[0]
    su = su_ref[...]

    stk = jnp.concatenate(
        [a[:, c * CH:(c + 1) * CH] for c in range(NCH)], axis=0)
    rest_stk = _mask_dot(stk, su)
    tot_stk = jnp.sum(stk, axis=1, keepdims=True)

    suffix = carry_ref[...]
    rests = [None] * NCH
    for c in reversed(range(NCH)):
        rests[c] = rest_stk[c * QH:(c + 1) * QH] + suffix
        suffix = suffix + tot_stk[c * QH:(c + 1) * QH]
    carry_ref[...] = suffix

    w = jnp.exp(jnp.concatenate(rests, axis=1))
    e = e_ref[...]
    wg = jax.lax.dot_general(e, w.astype(jnp.bfloat16), _MM,
                             preferred_element_type=jnp.float32)

    xb = x_ref[0].astype(jnp.bfloat16)
    bwb = (b_ref[0] * wg).astype(jnp.bfloat16)
    z = jax.lax.dot_general(
        xb, bwb, _KK, preferred_element_type=jnp.float32)

    @pl.when(s == 0)
    def _():
        acc_ref[...] = z

    @pl.when(s != 0)
    def _():
        acc_ref[...] = acc_ref[...] + z

    @pl.when(s == ns - 1)
    def _():
        acc = acc_ref[...]
        for hl in range(QH):
            o_ref[0, hl] = acc[hl * P_DIM:(hl + 1) * P_DIM,
                               hl * N_DIM:(hl + 1) * N_DIM]


def kernel(X, A, B_mat):
    B, S, H, P = X.shape
    N = B_mat.shape[-1]
    nq = H // QH
    ns = S // SBLK
    xw = QH * P
    bw = QH * N

    kk = jax.lax.broadcasted_iota(jnp.int32, (CH, CH), 0)
    ll = jax.lax.broadcasted_iota(jnp.int32, (CH, CH), 1)
    su = (kk > ll).astype(jnp.bfloat16)
    rr = jax.lax.broadcasted_iota(jnp.int32, (bw, QH), 0)
    hh = jax.lax.broadcasted_iota(jnp.int32, (bw, QH), 1)
    e = (rr // N == hh).astype(jnp.bfloat16)

    Xt = X.transpose(0, 2, 3, 1).reshape(B, H * P, S)
    Bt = B_mat.transpose(0, 2, 3, 1).reshape(B, H * N, S)
    At = A.transpose(0, 2, 1)

    out_full = pl.pallas_call(
        _fused_kernel,
        grid=(B * nq, ns),
        in_specs=[
            pl.BlockSpec((1, QH, SBLK), lambda i, s: (i // nq, i % nq, ns - 1 - s)),
            pl.BlockSpec((1, xw, SBLK), lambda i, s: (i // nq, i % nq, ns - 1 - s)),
            pl.BlockSpec((1, bw, SBLK), lambda i, s: (i // nq, i % nq, ns - 1 - s)),
            pl.BlockSpec((CH, CH), lambda i, s: (0, 0)),
            pl.BlockSpec((bw, QH), lambda i, s: (0, 0)),
        ],
        out_specs=pl.BlockSpec((1, QH, P, N), lambda i, s: (i, 0, 0, 0)),
        out_shape=jax.ShapeDtypeStruct((B * nq, QH, P, N), jnp.float32),
        scratch_shapes=[
            pltpu.VMEM((QH, 1), jnp.float32),
            pltpu.VMEM((xw, bw), jnp.float32),
        ],
        compiler_params=pltpu.CompilerParams(
            dimension_semantics=("parallel", "arbitrary")),
    )(At, Xt, Bt, su, e)

    return out_full.reshape(B, H, P, N)
